# baseline (speedup 1.0000x reference)
import functools

import jax
import jax.numpy as jnp
from jax import lax
from jax.experimental import pallas as pl
from jax.experimental.pallas import tpu as pltpu
from jax.experimental.pallas import tpu_sc as plsc

_K = 12
_KPAD = 16
_S_T = 512


def _topk_body(x_ref, x2_ref, xx_ref, xx2_ref, idx_ref):
    xb = x_ref[0]
    x2b = x2_ref[0]
    dot = lax.dot_general(x2b, xb, (((0,), (0,)), ((), ())),
                          preferred_element_type=jnp.float32)
    xx = xx_ref[0]
    xx2col = xx2_ref[0]
    dist = (2.0 * dot - xx) - xx2col
    s_t, p = dist.shape
    iota = lax.broadcasted_iota(jnp.int32, (s_t, p), 1)
    cols = []
    for _ in range(_K):
        m = jnp.max(dist, axis=1, keepdims=True)
        hit = dist == m
        idxv = jnp.min(jnp.where(hit, iota, p), axis=1, keepdims=True)
        cols.append(idxv)
        dist = jnp.where(iota == idxv, -jnp.inf, dist)
    cols.append(jnp.zeros((s_t, _KPAD - _K), jnp.int32))
    idx_ref[0] = jnp.concatenate(cols, axis=1)


def _topk_idx(x, x2):
    B, C, P = x.shape
    S = x2.shape[2]
    xx = jnp.sum(x**2, axis=1)[:, None, :]
    xx2 = jnp.sum(x2**2, axis=1)[:, :, None]
    grid = (B, S // _S_T)
    return pl.pallas_call(
        _topk_body,
        grid=grid,
        in_specs=[
            pl.BlockSpec((1, C, P), lambda b, s: (b, 0, 0)),
            pl.BlockSpec((1, C, _S_T), lambda b, s: (b, 0, s)),
            pl.BlockSpec((1, 1, P), lambda b, s: (b, 0, 0)),
            pl.BlockSpec((1, _S_T, 1), lambda b, s: (b, s, 0)),
        ],
        out_specs=pl.BlockSpec((1, _S_T, _KPAD), lambda b, s: (b, s, 0)),
        out_shape=jax.ShapeDtypeStruct((B, S, _KPAD), jnp.int32),
    )(x, x2, xx, xx2)


_G = 4


def _sc_gather(x, idx_flat, S):
    B, C, P = x.shape
    T = idx_flat.shape[1]
    info = plsc.get_sparse_core_info()
    nc, ns, L = info.num_cores, info.num_subcores, info.num_lanes
    nw = nc * ns
    ngroups = C // _G
    rounds = -(-ngroups // nw)
    x2d = x.reshape(B, C * P)
    mesh = plsc.VectorSubcoreMesh(core_axis_name="c", subcore_axis_name="s")

    @functools.partial(
        pl.kernel,
        mesh=mesh,
        out_type=jax.ShapeDtypeStruct((B, C, T), jnp.float32),
        scratch_types=[
            pltpu.VMEM((T,), jnp.int32),
            pltpu.VMEM((_G * P,), jnp.float32),
            pltpu.VMEM((_G, T), jnp.float32),
        ],
        compiler_params=pltpu.CompilerParams(needs_layout_passes=False),
    )
    def run(x_hbm, idx_hbm, out_hbm, idx_v, xg_v, out_v):
        wid = lax.axis_index("s") * nc + lax.axis_index("c")
        for b in range(B):
            pltpu.sync_copy(idx_hbm.at[b], idx_v)
            for i in range(rounds):
                g = i * nw + wid

                @pl.when(g < ngroups)
                def _():
                    c0 = g * _G
                    pltpu.sync_copy(x_hbm.at[b, pl.ds(c0 * P, _G * P)], xg_v)

                    @plsc.parallel_loop(0, T // L, unroll=16)
                    def body(t):
                        iv = idx_v[pl.ds(t * L, L)]
                        for r in range(_G):
                            out_v[r, pl.ds(t * L, L)] = plsc.load_gather(
                                xg_v, [iv + r * P])
                    pltpu.sync_copy(out_v, out_hbm.at[b, pl.ds(c0, _G)])

    return run(x2d, idx_flat)


def kernel(x, x2, k):
    B, C, P = x.shape
    S = x2.shape[2]
    idx = _topk_idx(x, x2)[:, :, :_K]
    idx_flat = jnp.transpose(idx, (0, 2, 1)).reshape(B, _K * S)
    out = _sc_gather(x, idx_flat, S)
    return out.reshape(B, C, S, _K)


# speedup vs baseline: 1.3226x; 1.3226x over previous
import functools

import jax
import jax.numpy as jnp
from jax import lax
from jax.experimental import pallas as pl
from jax.experimental.pallas import tpu as pltpu
from jax.experimental.pallas import tpu_sc as plsc

_K = 12
_KPAD = 16
_S_T = 256


def _topk_body(x_ref, x2_ref, xx_ref, xx2_ref, idx_ref):
    xb = x_ref[0]
    x2b = x2_ref[0]
    dot = lax.dot_general(x2b, xb, (((0,), (0,)), ((), ())),
                          preferred_element_type=jnp.float32)
    xx = xx_ref[0]
    xx2col = xx2_ref[0]
    dist = (2.0 * dot - xx) - xx2col
    s_t, p = dist.shape
    iota = lax.broadcasted_iota(jnp.int32, (s_t, p), 1)
    cols = []
    for _ in range(_K):
        m = jnp.max(dist, axis=1, keepdims=True)
        hit = dist == m
        idxv = jnp.min(jnp.where(hit, iota, p), axis=1, keepdims=True)
        cols.append(idxv)
        dist = jnp.where(iota == idxv, -jnp.inf, dist)
    cols.append(jnp.zeros((s_t, _KPAD - _K), jnp.int32))
    idx_ref[0] = jnp.concatenate(cols, axis=1)


def _topk_idx(x, x2):
    B, C, P = x.shape
    S = x2.shape[2]
    xx = jnp.sum(x**2, axis=1)[:, None, :]
    xx2 = jnp.sum(x2**2, axis=1)[:, :, None]
    grid = (B, S // _S_T)
    return pl.pallas_call(
        _topk_body,
        grid=grid,
        in_specs=[
            pl.BlockSpec((1, C, P), lambda b, s: (b, 0, 0)),
            pl.BlockSpec((1, C, _S_T), lambda b, s: (b, 0, s)),
            pl.BlockSpec((1, 1, P), lambda b, s: (b, 0, 0)),
            pl.BlockSpec((1, _S_T, 1), lambda b, s: (b, s, 0)),
        ],
        out_specs=pl.BlockSpec((1, _S_T, _KPAD), lambda b, s: (b, s, 0)),
        out_shape=jax.ShapeDtypeStruct((B, S, _KPAD), jnp.int32),
    )(x, x2, xx, xx2)


_G = 4


def _sc_gather(x, idx_jmajor, S):
    B, C, P = x.shape
    T = idx_jmajor.shape[1]
    info = plsc.get_sparse_core_info()
    nc, ns, L = info.num_cores, info.num_subcores, info.num_lanes
    nw = nc * ns
    ngroups = C // _G
    rounds = -(-ngroups // nw)
    sl = S // L
    x2d = x.reshape(B, C * P)
    mesh = plsc.VectorSubcoreMesh(core_axis_name="c", subcore_axis_name="s")

    @functools.partial(
        pl.kernel,
        mesh=mesh,
        out_type=jax.ShapeDtypeStruct((C, _K, B, S), jnp.float32),
        scratch_types=[
            pltpu.VMEM((T,), jnp.int32),
            pltpu.VMEM((_G * P,), jnp.float32),
            pltpu.VMEM((_G, _K, S), jnp.float32),
        ],
        compiler_params=pltpu.CompilerParams(needs_layout_passes=False),
    )
    def run(x_hbm, idx_hbm, out_hbm, idx_v, xg_v, out_v):
        wid = lax.axis_index("s") * nc + lax.axis_index("c")
        for b in range(B):
            pltpu.sync_copy(idx_hbm.at[b], idx_v)
            for i in range(rounds):
                g = i * nw + wid

                @pl.when(g < ngroups)
                def _():
                    c0 = g * _G
                    pltpu.sync_copy(x_hbm.at[b, pl.ds(c0 * P, _G * P)], xg_v)

                    @plsc.parallel_loop(0, T // L, unroll=8)
                    def body(t):
                        j = t // sl
                        s0 = (t % sl) * L
                        iv = idx_v[pl.ds(t * L, L)]
                        for r in range(_G):
                            out_v[r, j, pl.ds(s0, L)] = plsc.load_gather(
                                xg_v, [iv + r * P])
                    for r in range(_G):
                        pltpu.sync_copy(out_v.at[r],
                                        out_hbm.at[c0 + r, :, b])

    return run(x2d, idx_jmajor)


def kernel(x, x2, k):
    B, C, P = x.shape
    S = x2.shape[2]
    idx = _topk_idx(x, x2)[:, :, :_K]
    idx_flat = jnp.transpose(idx, (0, 2, 1)).reshape(B, S, _K)
    idx_jmajor = jnp.transpose(idx_flat, (0, 2, 1)).reshape(B, _K * S)
    out = _sc_gather(x, idx_jmajor, S)
    return jnp.transpose(out, (2, 0, 3, 1))
